# Initial kernel scaffold; baseline (speedup 1.0000x reference)
import functools

import jax
import jax.numpy as jnp
from jax import lax
from jax.experimental import pallas as pl
from jax.experimental.pallas import tpu as pltpu
from jax.experimental.pallas import tpu_sc as plsc

LPA = 128
MAX_PLY = 30
NUM_LS = 30
NUM_PA = 3
NUM_PATTERNS = 46
PATTERN_SIZE = 2187
TOTAL_FEATURES = NUM_PATTERNS * PATTERN_SIZE
BATCH = 16384

NC, NS = 2, 16
NW = NC * NS
BPW = BATCH // NW
E = 8
ROWS = E * NUM_PATTERNS
G = 4
GR = ROWS // G
NCHUNK = BPW // E
NVEC = LPA // 16

_mesh = plsc.VectorSubcoreMesh(core_axis_name="c", subcore_axis_name="s")


@functools.partial(
    pl.kernel,
    out_type=jax.ShapeDtypeStruct((BATCH,), jnp.float32),
    mesh=_mesh,
    scratch_types=[
        pltpu.VMEM((NCHUNK * G, GR), jnp.int32),
        pltpu.VMEM((ROWS, LPA), jnp.float32),
        pltpu.VMEM((ROWS, LPA), jnp.float32),
        pltpu.VMEM((BPW,), jnp.int32),
        pltpu.VMEM((NUM_PA, LPA), jnp.float32),
        pltpu.VMEM((NUM_LS, LPA), jnp.float32),
        pltpu.VMEM((32,), jnp.float32),
        pltpu.VMEM((BPW,), jnp.float32),
        pltpu.SemaphoreType.DMA,
        pltpu.SemaphoreType.DMA,
    ],
)
def _sc_forward(idx_hbm, ply_hbm, table_hbm, bias_hbm, w_hbm, ob_hbm, out_hbm,
                idx_all, rows0, rows1, ply_v, bias_v, w_v, ob_v, out_v,
                sem0, sem1):
    wid = lax.axis_index("s") * NC + lax.axis_index("c")
    base = wid * BPW

    pltpu.sync_copy(ply_hbm.at[pl.ds(base, BPW)], ply_v)
    pltpu.sync_copy(bias_hbm, bias_v)
    pltpu.sync_copy(w_hbm, w_v)
    pltpu.sync_copy(ob_hbm, ob_v)
    pltpu.sync_copy(idx_hbm.at[pl.ds(wid * (NCHUNK * G), NCHUNK * G)], idx_all)

    def issue(c, rows_buf, sem):
        for g in range(G):
            pltpu.async_copy(table_hbm.at[idx_all.at[c * G + g]],
                             rows_buf.at[pl.ds(g * GR, GR)], sem)

    def drain(c, rows_buf, sem):
        for g in range(G):
            pltpu.make_async_copy(table_hbm.at[idx_all.at[c * G + g]],
                                  rows_buf.at[pl.ds(g * GR, GR)], sem).wait()

    def compute(c, rows_buf):
        def elem(e, _):
            b = c * E + e
            p = ply_v[b]
            bkt = p // (MAX_PLY // NUM_PA)
            acc = tuple(bias_v[bkt, pl.ds(k * 16, 16)] for k in range(NVEC))

            def row2(j, acc):
                r0 = e * NUM_PATTERNS + 2 * j
                a = tuple(acc[k] + rows_buf[r0, pl.ds(k * 16, 16)]
                          for k in range(NVEC))
                return tuple(a[k] + rows_buf[r0 + 1, pl.ds(k * 16, 16)]
                             for k in range(NVEC))

            acc = lax.fori_loop(0, NUM_PATTERNS // 2, row2, acc)
            t = jnp.zeros((16,), jnp.float32)
            for k in range(NVEC):
                xk = jnp.clip(acc[k], 0.0, 1.0)
                t = t + xk * w_v[p, pl.ds(k * 16, 16)]
            out_v[b] = jnp.sum(t) + ob_v[p]
            return 0

        lax.fori_loop(0, E, elem, 0)

    issue(0, rows0, sem0)

    def body2(i, _):
        c0 = 2 * i
        issue(c0 + 1, rows1, sem1)
        drain(c0, rows0, sem0)
        compute(c0, rows0)

        @pl.when(c0 + 2 < NCHUNK)
        def _():
            issue(c0 + 2, rows0, sem0)

        drain(c0 + 1, rows1, sem1)
        compute(c0 + 1, rows1)
        return 0

    lax.fori_loop(0, NCHUNK // 2, body2, 0)
    pltpu.sync_copy(out_v, out_hbm.at[pl.ds(base, BPW)])


@jax.jit
def kernel(feature_indices, ply, pa_weight, pa_bias, out_weight, out_bias,
           feature_offsets):
    bkt = ply // (MAX_PLY // NUM_PA)
    idx = (feature_indices + feature_offsets[None, :]
           + bkt[:, None] * TOTAL_FEATURES).astype(jnp.int32)
    idx2 = idx.reshape(-1, GR)
    table = pa_weight.reshape(NUM_PA * TOTAL_FEATURES, LPA)
    w = out_weight.reshape(NUM_LS, LPA)
    ob = jnp.pad(out_bias.reshape(NUM_LS), (0, 32 - NUM_LS))
    out = _sc_forward(idx2, ply, table, pa_bias, w, ob)
    return out.reshape(BATCH, 1)


# speedup vs baseline: 2.1983x; 2.1983x over previous
import functools

import jax
import jax.numpy as jnp
from jax import lax
from jax.experimental import pallas as pl
from jax.experimental.pallas import tpu as pltpu
from jax.experimental.pallas import tpu_sc as plsc

LPA = 128
MAX_PLY = 30
NUM_LS = 30
NUM_PA = 3
NUM_PATTERNS = 46
PATTERN_SIZE = 2187
TOTAL_FEATURES = NUM_PATTERNS * PATTERN_SIZE
BATCH = 16384

NC, NS = 2, 16
NW = NC * NS
BPW = BATCH // NW
EG = 16
NG = BPW // EG
HALF = NUM_PATTERNS // 2
ROWS = HALF * EG
G = 4
GR = ROWS // G
NSUB = 2 * NG
NVEC = LPA // 16

_mesh = plsc.VectorSubcoreMesh(core_axis_name="c", subcore_axis_name="s")


@functools.partial(
    pl.kernel,
    out_type=jax.ShapeDtypeStruct((BATCH, LPA), jnp.float32),
    mesh=_mesh,
    scratch_types=[
        pltpu.VMEM((G, GR), jnp.int32),
        pltpu.VMEM((G, GR), jnp.int32),
        pltpu.VMEM((ROWS, LPA), jnp.float32),
        pltpu.VMEM((ROWS, LPA), jnp.float32),
        pltpu.VMEM((EG, LPA), jnp.float32),
        pltpu.SemaphoreType.DMA,
        pltpu.SemaphoreType.DMA,
        pltpu.SemaphoreType.DMA,
        pltpu.SemaphoreType.DMA,
        pltpu.SemaphoreType.DMA,
    ],
)
def _sc_gather_sum(idx_hbm, table_hbm, outx_hbm,
                   idx0, idx1, rows0, rows1, xbuf,
                   sem0, sem1, semi0, semi1, semx):
    wid = lax.axis_index("s") * NC + lax.axis_index("c")
    base = wid * BPW
    ibase = wid * (NSUB * G)

    def fetch_idx(sc, idx_buf, sem):
        pltpu.async_copy(idx_hbm.at[pl.ds(ibase + sc * G, G)], idx_buf, sem)

    def wait_idx(sc, idx_buf, sem):
        pltpu.make_async_copy(idx_hbm.at[pl.ds(ibase + sc * G, G)],
                              idx_buf, sem).wait()

    def issue(idx_buf, rows_buf, sem):
        for g in range(G):
            pltpu.async_copy(table_hbm.at[idx_buf.at[g]],
                             rows_buf.at[pl.ds(g * GR, GR)], sem)

    def drain(idx_buf, rows_buf, sem):
        for g in range(G):
            pltpu.make_async_copy(table_hbm.at[idx_buf.at[g]],
                                  rows_buf.at[pl.ds(g * GR, GR)], sem).wait()

    def xslice(g):
        return outx_hbm.at[pl.ds(base + g * EG, EG)]

    def accum_rows(rows_buf, e, acc):
        def rowb(j, acc):
            r = j * EG + e
            return tuple(acc[k] + rows_buf[r, pl.ds(k * 16, 16)]
                         for k in range(NVEC))
        return lax.fori_loop(0, HALF, rowb, acc)

    def pass1(rows_buf):
        zero = tuple(jnp.zeros((16,), jnp.float32) for _ in range(NVEC))
        for e in range(EG):
            acc = accum_rows(rows_buf, e, zero)
            for k in range(NVEC):
                xbuf[e, pl.ds(k * 16, 16)] = acc[k]

    def pass2(g, rows_buf):
        for e in range(EG):
            zero = tuple(jnp.zeros((16,), jnp.float32) for _ in range(NVEC))
            acc = accum_rows(rows_buf, e, zero)
            for k in range(NVEC):
                xbuf[e, pl.ds(k * 16, 16)] = (xbuf[e, pl.ds(k * 16, 16)]
                                              + acc[k])
        pltpu.async_copy(xbuf, xslice(g), semx)

    fetch_idx(0, idx0, semi0)
    wait_idx(0, idx0, semi0)
    issue(idx0, rows0, sem0)
    fetch_idx(1, idx1, semi1)

    def group_body(g, _):
        s0 = 2 * g
        wait_idx(s0 + 1, idx1, semi1)
        issue(idx1, rows1, sem1)
        drain(idx0, rows0, sem0)

        @pl.when(g + 1 < NG)
        def _():
            fetch_idx(s0 + 2, idx0, semi0)

        @pl.when(g > 0)
        def _():
            pltpu.make_async_copy(xbuf, xslice(g - 1), semx).wait()

        pass1(rows0)

        @pl.when(g + 1 < NG)
        def _():
            wait_idx(s0 + 2, idx0, semi0)
            issue(idx0, rows0, sem0)

        drain(idx1, rows1, sem1)

        @pl.when(g + 1 < NG)
        def _():
            fetch_idx(s0 + 3, idx1, semi1)

        pass2(g, rows1)
        return 0

    lax.fori_loop(0, NG, group_body, 0)
    pltpu.make_async_copy(xbuf, xslice(NG - 1), semx).wait()


TB = 512
NB = BATCH // TB


def _tc_head_body(ply_ref, x_ref, bias_ref, w_ref, ob_ref, out_ref):
    x = x_ref[...]
    plyv = ply_ref[0, 0, :]
    bkt = lax.div(plyv, MAX_PLY // NUM_PA)
    oh_pa = jnp.asarray(
        bkt[:, None] == lax.broadcasted_iota(jnp.int32, (TB, 8), 1),
        jnp.float32)
    brow = lax.dot_general(oh_pa, bias_ref[...],
                           (((1,), (0,)), ((), ())),
                           preferred_element_type=jnp.float32)
    x_pa = jnp.clip(x + brow, 0.0, 1.0)
    p = lax.dot_general(x_pa, w_ref[...],
                        (((1,), (1,)), ((), ())),
                        preferred_element_type=jnp.float32)
    oh_ls = jnp.asarray(
        plyv[:, None] == lax.broadcasted_iota(jnp.int32, (TB, 32), 1),
        jnp.float32)
    out_ref[0, 0, :] = jnp.sum((p + ob_ref[...]) * oh_ls, axis=1)


_tc_head = pl.pallas_call(
    _tc_head_body,
    grid=(NB,),
    in_specs=[
        pl.BlockSpec((1, 1, TB), lambda i: (i, 0, 0)),
        pl.BlockSpec((TB, LPA), lambda i: (i, 0)),
        pl.BlockSpec((8, LPA), lambda i: (0, 0)),
        pl.BlockSpec((32, LPA), lambda i: (0, 0)),
        pl.BlockSpec((1, 32), lambda i: (0, 0)),
    ],
    out_specs=pl.BlockSpec((1, 1, TB), lambda i: (i, 0, 0)),
    out_shape=jax.ShapeDtypeStruct((NB, 1, TB), jnp.float32),
)


@jax.jit
def kernel(feature_indices, ply, pa_weight, pa_bias, out_weight, out_bias,
           feature_offsets):
    bkt = ply // (MAX_PLY // NUM_PA)
    idx = (feature_indices + feature_offsets[None, :]
           + bkt[:, None] * TOTAL_FEATURES).astype(jnp.int32)
    idx5 = idx.reshape(NW, NG, EG, 2, HALF).transpose(0, 1, 3, 4, 2)
    idx2 = idx5.reshape(-1, GR)
    table = pa_weight.reshape(NUM_PA * TOTAL_FEATURES, LPA)
    x_sum = _sc_gather_sum(idx2, table)

    ply3 = ply.reshape(NB, 1, TB)
    biasp = jnp.pad(pa_bias, ((0, 8 - NUM_PA), (0, 0)))
    wp = jnp.pad(out_weight.reshape(NUM_LS, LPA), ((0, 32 - NUM_LS), (0, 0)))
    obp = jnp.pad(out_bias.reshape(1, NUM_LS), ((0, 0), (0, 32 - NUM_LS)))
    out = _tc_head(ply3, x_sum, biasp, wp, obp)
    return out.reshape(BATCH, 1)


# speedup vs baseline: 2.7664x; 1.2584x over previous
import functools

import jax
import jax.numpy as jnp
from jax import lax
from jax.experimental import pallas as pl
from jax.experimental.pallas import tpu as pltpu
from jax.experimental.pallas import tpu_sc as plsc

LPA = 128
MAX_PLY = 30
NUM_LS = 30
NUM_PA = 3
NUM_PATTERNS = 46
PATTERN_SIZE = 2187
TOTAL_FEATURES = NUM_PATTERNS * PATTERN_SIZE
BATCH = 16384

NC, NS = 2, 16
NW = NC * NS
BPW = BATCH // NW
EC = 8
ROWS = EC * NUM_PATTERNS
G = 4
GR = ROWS // G
NCHUNK = BPW // EC
NVEC = LPA // 16

_mesh = plsc.VectorSubcoreMesh(core_axis_name="c", subcore_axis_name="s")


@functools.partial(
    pl.kernel,
    out_type=jax.ShapeDtypeStruct((BATCH, LPA), jnp.float32),
    mesh=_mesh,
    scratch_types=[
        pltpu.VMEM((G, GR), jnp.int32),
        pltpu.VMEM((G, GR), jnp.int32),
        pltpu.VMEM((ROWS, LPA), jnp.float32),
        pltpu.VMEM((ROWS, LPA), jnp.float32),
        pltpu.VMEM((EC, LPA), jnp.float32),
        pltpu.VMEM((EC, LPA), jnp.float32),
        pltpu.SemaphoreType.DMA,
        pltpu.SemaphoreType.DMA,
        pltpu.SemaphoreType.DMA,
        pltpu.SemaphoreType.DMA,
        pltpu.SemaphoreType.DMA,
        pltpu.SemaphoreType.DMA,
    ],
)
def _sc_gather_sum(idx_hbm, table_hbm, outx_hbm,
                   idx0, idx1, rows0, rows1, xbuf0, xbuf1,
                   sem0, sem1, semi0, semi1, semx0, semx1):
    wid = lax.axis_index("s") * NC + lax.axis_index("c")
    base = wid * BPW
    ibase = wid * (NCHUNK * G)

    def fetch_idx(c, idx_buf, sem):
        pltpu.async_copy(idx_hbm.at[pl.ds(ibase + c * G, G)], idx_buf, sem)

    def wait_idx(c, idx_buf, sem):
        pltpu.make_async_copy(idx_hbm.at[pl.ds(ibase + c * G, G)],
                              idx_buf, sem).wait()

    def issue(idx_buf, rows_buf, sem):
        for g in range(G):
            pltpu.async_copy(table_hbm.at[idx_buf.at[g]],
                             rows_buf.at[pl.ds(g * GR, GR)], sem)

    def drain(idx_buf, rows_buf, sem):
        for g in range(G):
            pltpu.make_async_copy(table_hbm.at[idx_buf.at[g]],
                                  rows_buf.at[pl.ds(g * GR, GR)], sem).wait()

    def xslice(c):
        return outx_hbm.at[pl.ds(base + c * EC, EC)]

    def compute(rows_buf, xbuf):
        for e in range(EC):
            def rowb(j, acc):
                r = e * NUM_PATTERNS + 2 * j
                a = tuple(acc[k] + rows_buf[r, pl.ds(k * 16, 16)]
                          for k in range(NVEC))
                return tuple(a[k] + rows_buf[r + 1, pl.ds(k * 16, 16)]
                             for k in range(NVEC))
            zero = tuple(jnp.zeros((16,), jnp.float32) for _ in range(NVEC))
            acc = lax.fori_loop(0, NUM_PATTERNS // 2, rowb, zero)
            for k in range(NVEC):
                xbuf[e, pl.ds(k * 16, 16)] = acc[k]

    fetch_idx(0, idx0, semi0)
    wait_idx(0, idx0, semi0)
    issue(idx0, rows0, sem0)
    fetch_idx(1, idx1, semi1)

    def body2(i, _):
        c0 = 2 * i
        c1 = c0 + 1
        wait_idx(c1, idx1, semi1)
        issue(idx1, rows1, sem1)
        drain(idx0, rows0, sem0)

        @pl.when(i + 1 < NCHUNK // 2)
        def _():
            fetch_idx(c0 + 2, idx0, semi0)

        @pl.when(i > 0)
        def _():
            pltpu.make_async_copy(xbuf0, xslice(c0 - 2), semx0).wait()

        compute(rows0, xbuf0)
        pltpu.async_copy(xbuf0, xslice(c0), semx0)

        @pl.when(i + 1 < NCHUNK // 2)
        def _():
            wait_idx(c0 + 2, idx0, semi0)
            issue(idx0, rows0, sem0)

        drain(idx1, rows1, sem1)

        @pl.when(i + 1 < NCHUNK // 2)
        def _():
            fetch_idx(c1 + 2, idx1, semi1)

        @pl.when(i > 0)
        def _():
            pltpu.make_async_copy(xbuf1, xslice(c1 - 2), semx1).wait()

        compute(rows1, xbuf1)
        pltpu.async_copy(xbuf1, xslice(c1), semx1)
        return 0

    lax.fori_loop(0, NCHUNK // 2, body2, 0)
    pltpu.make_async_copy(xbuf0, xslice(NCHUNK - 2), semx0).wait()
    pltpu.make_async_copy(xbuf1, xslice(NCHUNK - 1), semx1).wait()


TB = 512
NB = BATCH // TB


def _tc_head_body(ply_ref, x_ref, bias_ref, w_ref, ob_ref, out_ref):
    x = x_ref[...]
    plyv = ply_ref[0, 0, :]
    bkt = lax.div(plyv, MAX_PLY // NUM_PA)
    oh_pa = jnp.asarray(
        bkt[:, None] == lax.broadcasted_iota(jnp.int32, (TB, 8), 1),
        jnp.float32)
    brow = lax.dot_general(oh_pa, bias_ref[...],
                           (((1,), (0,)), ((), ())),
                           preferred_element_type=jnp.float32)
    x_pa = jnp.clip(x + brow, 0.0, 1.0)
    p = lax.dot_general(x_pa, w_ref[...],
                        (((1,), (1,)), ((), ())),
                        preferred_element_type=jnp.float32)
    oh_ls = jnp.asarray(
        plyv[:, None] == lax.broadcasted_iota(jnp.int32, (TB, 32), 1),
        jnp.float32)
    out_ref[0, 0, :] = jnp.sum((p + ob_ref[...]) * oh_ls, axis=1)


_tc_head = pl.pallas_call(
    _tc_head_body,
    grid=(NB,),
    in_specs=[
        pl.BlockSpec((1, 1, TB), lambda i: (i, 0, 0)),
        pl.BlockSpec((TB, LPA), lambda i: (i, 0)),
        pl.BlockSpec((8, LPA), lambda i: (0, 0)),
        pl.BlockSpec((32, LPA), lambda i: (0, 0)),
        pl.BlockSpec((1, 32), lambda i: (0, 0)),
    ],
    out_specs=pl.BlockSpec((1, 1, TB), lambda i: (i, 0, 0)),
    out_shape=jax.ShapeDtypeStruct((NB, 1, TB), jnp.float32),
)


@jax.jit
def kernel(feature_indices, ply, pa_weight, pa_bias, out_weight, out_bias,
           feature_offsets):
    bkt = ply // (MAX_PLY // NUM_PA)
    idx = (feature_indices + feature_offsets[None, :]
           + bkt[:, None] * TOTAL_FEATURES).astype(jnp.int32)
    idx2 = idx.reshape(-1, GR)
    table = pa_weight.reshape(NUM_PA * TOTAL_FEATURES, LPA)
    x_sum = _sc_gather_sum(idx2, table)

    ply3 = ply.reshape(NB, 1, TB)
    biasp = jnp.pad(pa_bias, ((0, 8 - NUM_PA), (0, 0)))
    wp = jnp.pad(out_weight.reshape(NUM_LS, LPA), ((0, 32 - NUM_LS), (0, 0)))
    obp = jnp.pad(out_bias.reshape(1, NUM_LS), ((0, 0), (0, 32 - NUM_LS)))
    out = _tc_head(ply3, x_sum, biasp, wp, obp)
    return out.reshape(BATCH, 1)


# speedup vs baseline: 2.8614x; 1.0343x over previous
import functools

import jax
import jax.numpy as jnp
from jax import lax
from jax.experimental import pallas as pl
from jax.experimental.pallas import tpu as pltpu
from jax.experimental.pallas import tpu_sc as plsc

LPA = 128
MAX_PLY = 30
NUM_LS = 30
NUM_PA = 3
NUM_PATTERNS = 46
PATTERN_SIZE = 2187
TOTAL_FEATURES = NUM_PATTERNS * PATTERN_SIZE
BATCH = 16384

NC, NS = 2, 16
NW = NC * NS
BPW = BATCH // NW
EC = 8
ROWS = EC * NUM_PATTERNS
G = 4
GR = ROWS // G
NCHUNK = BPW // EC
NVEC = LPA // 16

_mesh = plsc.VectorSubcoreMesh(core_axis_name="c", subcore_axis_name="s")


@functools.partial(
    pl.kernel,
    out_type=jax.ShapeDtypeStruct((BATCH, LPA), jnp.float32),
    mesh=_mesh,
    scratch_types=[
        pltpu.VMEM((EC, NUM_PATTERNS), jnp.int32),
        pltpu.VMEM((EC, NUM_PATTERNS), jnp.int32),
        pltpu.VMEM((ROWS, LPA), jnp.float32),
        pltpu.VMEM((ROWS, LPA), jnp.float32),
        pltpu.VMEM((EC, LPA), jnp.float32),
        pltpu.VMEM((EC, LPA), jnp.float32),
        pltpu.SemaphoreType.DMA,
        pltpu.SemaphoreType.DMA,
        pltpu.SemaphoreType.DMA,
        pltpu.SemaphoreType.DMA,
        pltpu.SemaphoreType.DMA,
        pltpu.SemaphoreType.DMA,
    ],
)
def _sc_gather_sum(idx_hbm, table_hbm, outx_hbm,
                   idx0, idx1, rows0, rows1, xbuf0, xbuf1,
                   sem0, sem1, semi0, semi1, semx0, semx1):
    wid = lax.axis_index("s") * NC + lax.axis_index("c")
    base = wid * BPW

    def fetch_idx(c, idx_buf, sem):
        pltpu.async_copy(idx_hbm.at[pl.ds(base + c * EC, EC)], idx_buf, sem)

    def wait_idx(c, idx_buf, sem):
        pltpu.make_async_copy(idx_hbm.at[pl.ds(base + c * EC, EC)],
                              idx_buf, sem).wait()

    def issue(idx_buf, rows_buf, sem):
        for e in range(EC):
            pltpu.async_copy(table_hbm.at[idx_buf.at[e]],
                             rows_buf.at[pl.ds(e * NUM_PATTERNS,
                                               NUM_PATTERNS)], sem)

    def drain(idx_buf, rows_buf, sem):
        for e in range(EC):
            pltpu.make_async_copy(table_hbm.at[idx_buf.at[e]],
                                  rows_buf.at[pl.ds(e * NUM_PATTERNS,
                                                    NUM_PATTERNS)],
                                  sem).wait()

    def xslice(c):
        return outx_hbm.at[pl.ds(base + c * EC, EC)]

    def compute(rows_buf, xbuf):
        for e in range(EC):
            def rowb(j, acc):
                r = e * NUM_PATTERNS + 2 * j
                a = tuple(acc[k] + rows_buf[r, pl.ds(k * 16, 16)]
                          for k in range(NVEC))
                return tuple(a[k] + rows_buf[r + 1, pl.ds(k * 16, 16)]
                             for k in range(NVEC))
            zero = tuple(jnp.zeros((16,), jnp.float32) for _ in range(NVEC))
            acc = lax.fori_loop(0, NUM_PATTERNS // 2, rowb, zero)
            for k in range(NVEC):
                xbuf[e, pl.ds(k * 16, 16)] = acc[k]

    fetch_idx(0, idx0, semi0)
    wait_idx(0, idx0, semi0)
    issue(idx0, rows0, sem0)
    fetch_idx(1, idx1, semi1)

    def body2(i, _):
        c0 = 2 * i
        c1 = c0 + 1
        wait_idx(c1, idx1, semi1)
        issue(idx1, rows1, sem1)
        drain(idx0, rows0, sem0)

        @pl.when(i + 1 < NCHUNK // 2)
        def _():
            fetch_idx(c0 + 2, idx0, semi0)

        @pl.when(i > 0)
        def _():
            pltpu.make_async_copy(xbuf0, xslice(c0 - 2), semx0).wait()

        compute(rows0, xbuf0)
        pltpu.async_copy(xbuf0, xslice(c0), semx0)

        @pl.when(i + 1 < NCHUNK // 2)
        def _():
            wait_idx(c0 + 2, idx0, semi0)
            issue(idx0, rows0, sem0)

        drain(idx1, rows1, sem1)

        @pl.when(i + 1 < NCHUNK // 2)
        def _():
            fetch_idx(c1 + 2, idx1, semi1)

        @pl.when(i > 0)
        def _():
            pltpu.make_async_copy(xbuf1, xslice(c1 - 2), semx1).wait()

        compute(rows1, xbuf1)
        pltpu.async_copy(xbuf1, xslice(c1), semx1)
        return 0

    lax.fori_loop(0, NCHUNK // 2, body2, 0)
    pltpu.make_async_copy(xbuf0, xslice(NCHUNK - 2), semx0).wait()
    pltpu.make_async_copy(xbuf1, xslice(NCHUNK - 1), semx1).wait()


TB = 2048
NB = BATCH // TB


def _tc_head_body(ply_ref, x_ref, bias_ref, w_ref, ob_ref, out_ref):
    x = x_ref[...]
    plyv = ply_ref[0, 0, :]
    bkt = lax.div(plyv, MAX_PLY // NUM_PA)
    oh_pa = jnp.asarray(
        bkt[:, None] == lax.broadcasted_iota(jnp.int32, (TB, 8), 1),
        jnp.float32)
    brow = lax.dot_general(oh_pa, bias_ref[...],
                           (((1,), (0,)), ((), ())),
                           preferred_element_type=jnp.float32)
    x_pa = jnp.clip(x + brow, 0.0, 1.0)
    p = lax.dot_general(x_pa, w_ref[...],
                        (((1,), (1,)), ((), ())),
                        preferred_element_type=jnp.float32)
    oh_ls = jnp.asarray(
        plyv[:, None] == lax.broadcasted_iota(jnp.int32, (TB, 32), 1),
        jnp.float32)
    out_ref[0, 0, :] = jnp.sum((p + ob_ref[...]) * oh_ls, axis=1)


_tc_head = pl.pallas_call(
    _tc_head_body,
    grid=(NB,),
    in_specs=[
        pl.BlockSpec((1, 1, TB), lambda i: (i, 0, 0)),
        pl.BlockSpec((TB, LPA), lambda i: (i, 0)),
        pl.BlockSpec((8, LPA), lambda i: (0, 0)),
        pl.BlockSpec((32, LPA), lambda i: (0, 0)),
        pl.BlockSpec((1, 32), lambda i: (0, 0)),
    ],
    out_specs=pl.BlockSpec((1, 1, TB), lambda i: (i, 0, 0)),
    out_shape=jax.ShapeDtypeStruct((NB, 1, TB), jnp.float32),
)


@jax.jit
def kernel(feature_indices, ply, pa_weight, pa_bias, out_weight, out_bias,
           feature_offsets):
    bkt = ply // (MAX_PLY // NUM_PA)
    idx = (feature_indices + feature_offsets[None, :]
           + bkt[:, None] * TOTAL_FEATURES).astype(jnp.int32)
    table = pa_weight.reshape(NUM_PA * TOTAL_FEATURES, LPA)
    x_sum = _sc_gather_sum(idx, table)

    ply3 = ply.reshape(NB, 1, TB)
    biasp = jnp.pad(pa_bias, ((0, 8 - NUM_PA), (0, 0)))
    wp = jnp.pad(out_weight.reshape(NUM_LS, LPA), ((0, 32 - NUM_LS), (0, 0)))
    obp = jnp.pad(out_bias.reshape(1, NUM_LS), ((0, 0), (0, 32 - NUM_LS)))
    out = _tc_head(ply3, x_sum, biasp, wp, obp)
    return out.reshape(BATCH, 1)


# speedup vs baseline: 4.5616x; 1.5942x over previous
import functools

import jax
import jax.numpy as jnp
from jax import lax
from jax.experimental import pallas as pl
from jax.experimental.pallas import tpu as pltpu
from jax.experimental.pallas import tpu_sc as plsc

LPA = 128
MAX_PLY = 30
NUM_LS = 30
NUM_PA = 3
NUM_PATTERNS = 46
PATTERN_SIZE = 2187
TOTAL_FEATURES = NUM_PATTERNS * PATTERN_SIZE
BATCH = 16384

NC, NS = 2, 16
NW = NC * NS
BPW = BATCH // NW
EC = 8
ROWS = EC * NUM_PATTERNS
G = 4
GR = ROWS // G
NCHUNK = BPW // EC
NVEC = LPA // 16

_mesh = plsc.VectorSubcoreMesh(core_axis_name="c", subcore_axis_name="s")


@functools.partial(
    pl.kernel,
    out_type=jax.ShapeDtypeStruct((BATCH, LPA), jnp.float32),
    mesh=_mesh,
    scratch_types=[
        pltpu.VMEM((EC, NUM_PATTERNS), jnp.int32),
        pltpu.VMEM((EC, NUM_PATTERNS), jnp.int32),
        pltpu.VMEM((ROWS, LPA), jnp.float32),
        pltpu.VMEM((ROWS, LPA), jnp.float32),
        pltpu.VMEM((EC, LPA), jnp.float32),
        pltpu.VMEM((EC, LPA), jnp.float32),
        pltpu.VMEM((BPW + 16, ), jnp.int32),
        pltpu.SemaphoreType.DMA,
        pltpu.SemaphoreType.DMA,
        pltpu.SemaphoreType.DMA,
        pltpu.SemaphoreType.DMA,
        pltpu.SemaphoreType.DMA,
        pltpu.SemaphoreType.DMA,
    ],
)
def _sc_gather_sum(idx_hbm, bkt_hbm, table_hbm, outx_hbm,
                   idx0, idx1, rows0, rows1, xbuf0, xbuf1, bkt_v,
                   sem0, sem1, semi0, semi1, semx0, semx1):
    wid = lax.axis_index("s") * NC + lax.axis_index("c")
    base = wid * BPW
    pltpu.sync_copy(bkt_hbm.at[pl.ds(base, BPW)], bkt_v.at[pl.ds(0, BPW)])

    def fetch_idx(c, idx_buf, sem):
        pltpu.async_copy(idx_hbm.at[pl.ds(base + c * EC, EC)], idx_buf, sem)

    def wait_idx(c, idx_buf, sem):
        pltpu.make_async_copy(idx_hbm.at[pl.ds(base + c * EC, EC)],
                              idx_buf, sem).wait()

    def issue(c, idx_buf, rows_buf, sem):
        bv = bkt_v[pl.ds(c * EC, 16)]
        for e in range(EC):
            b = bv[e]
            pltpu.async_copy(table_hbm.at[b].at[idx_buf.at[e]],
                             rows_buf.at[pl.ds(e * NUM_PATTERNS,
                                               NUM_PATTERNS)], sem)

    def drain(c, idx_buf, rows_buf, sem):
        bv = bkt_v[pl.ds(c * EC, 16)]
        for e in range(EC):
            b = bv[e]
            pltpu.make_async_copy(table_hbm.at[b].at[idx_buf.at[e]],
                                  rows_buf.at[pl.ds(e * NUM_PATTERNS,
                                                    NUM_PATTERNS)],
                                  sem).wait()

    def xslice(c):
        return outx_hbm.at[pl.ds(base + c * EC, EC)]

    def compute(rows_buf, xbuf):
        for e in range(EC):
            def rowb(j, acc):
                r = e * NUM_PATTERNS + 2 * j
                a = tuple(acc[k] + rows_buf[r, pl.ds(k * 16, 16)]
                          for k in range(NVEC))
                return tuple(a[k] + rows_buf[r + 1, pl.ds(k * 16, 16)]
                             for k in range(NVEC))
            zero = tuple(jnp.zeros((16,), jnp.float32) for _ in range(NVEC))
            acc = lax.fori_loop(0, NUM_PATTERNS // 2, rowb, zero)
            for k in range(NVEC):
                xbuf[e, pl.ds(k * 16, 16)] = acc[k]

    fetch_idx(0, idx0, semi0)
    wait_idx(0, idx0, semi0)
    issue(0, idx0, rows0, sem0)
    fetch_idx(1, idx1, semi1)

    def body2(i, _):
        c0 = 2 * i
        c1 = c0 + 1
        wait_idx(c1, idx1, semi1)
        issue(c1, idx1, rows1, sem1)
        drain(c0, idx0, rows0, sem0)

        @pl.when(i + 1 < NCHUNK // 2)
        def _():
            fetch_idx(c0 + 2, idx0, semi0)

        @pl.when(i > 0)
        def _():
            pltpu.make_async_copy(xbuf0, xslice(c0 - 2), semx0).wait()

        compute(rows0, xbuf0)
        pltpu.async_copy(xbuf0, xslice(c0), semx0)

        @pl.when(i + 1 < NCHUNK // 2)
        def _():
            wait_idx(c0 + 2, idx0, semi0)
            issue(c0 + 2, idx0, rows0, sem0)

        drain(c1, idx1, rows1, sem1)

        @pl.when(i + 1 < NCHUNK // 2)
        def _():
            fetch_idx(c1 + 2, idx1, semi1)

        @pl.when(i > 0)
        def _():
            pltpu.make_async_copy(xbuf1, xslice(c1 - 2), semx1).wait()

        compute(rows1, xbuf1)
        pltpu.async_copy(xbuf1, xslice(c1), semx1)
        return 0

    lax.fori_loop(0, NCHUNK // 2, body2, 0)
    pltpu.make_async_copy(xbuf0, xslice(NCHUNK - 2), semx0).wait()
    pltpu.make_async_copy(xbuf1, xslice(NCHUNK - 1), semx1).wait()


TB = 2048
NB = BATCH // TB


def _tc_head_body(ply_ref, x_ref, bias_ref, w_ref, ob_ref, out_ref):
    x = x_ref[...]
    plyv = ply_ref[0, 0, :]
    bkt = lax.div(plyv, MAX_PLY // NUM_PA)
    oh_pa = jnp.asarray(
        bkt[:, None] == lax.broadcasted_iota(jnp.int32, (TB, 8), 1),
        jnp.float32)
    brow = lax.dot_general(oh_pa, bias_ref[...],
                           (((1,), (0,)), ((), ())),
                           preferred_element_type=jnp.float32)
    x_pa = jnp.clip(x + brow, 0.0, 1.0)
    p = lax.dot_general(x_pa, w_ref[...],
                        (((1,), (1,)), ((), ())),
                        preferred_element_type=jnp.float32)
    oh_ls = jnp.asarray(
        plyv[:, None] == lax.broadcasted_iota(jnp.int32, (TB, 32), 1),
        jnp.float32)
    out_ref[0, 0, :] = jnp.sum((p + ob_ref[...]) * oh_ls, axis=1)


_tc_head = pl.pallas_call(
    _tc_head_body,
    grid=(NB,),
    in_specs=[
        pl.BlockSpec((1, 1, TB), lambda i: (i, 0, 0)),
        pl.BlockSpec((TB, LPA), lambda i: (i, 0)),
        pl.BlockSpec((8, LPA), lambda i: (0, 0)),
        pl.BlockSpec((32, LPA), lambda i: (0, 0)),
        pl.BlockSpec((1, 32), lambda i: (0, 0)),
    ],
    out_specs=pl.BlockSpec((1, 1, TB), lambda i: (i, 0, 0)),
    out_shape=jax.ShapeDtypeStruct((NB, 1, TB), jnp.float32),
)


@jax.jit
def kernel(feature_indices, ply, pa_weight, pa_bias, out_weight, out_bias,
           feature_offsets):
    bkt = ply // (MAX_PLY // NUM_PA)
    idx = (feature_indices + feature_offsets[None, :]).astype(jnp.int32)
    x_sum = _sc_gather_sum(idx, bkt, pa_weight)

    ply3 = ply.reshape(NB, 1, TB)
    biasp = jnp.pad(pa_bias, ((0, 8 - NUM_PA), (0, 0)))
    wp = jnp.pad(out_weight.reshape(NUM_LS, LPA), ((0, 32 - NUM_LS), (0, 0)))
    obp = jnp.pad(out_bias.reshape(1, NUM_LS), ((0, 0), (0, 32 - NUM_LS)))
    out = _tc_head(ply3, x_sum, biasp, wp, obp)
    return out.reshape(BATCH, 1)
